# baseline (speedup 1.0000x reference)
import numpy as np
import jax
import jax.numpy as jnp
from jax import lax
from jax.experimental import pallas as pl
from jax.experimental.pallas import tpu as pltpu
from jax.experimental.pallas import tpu_sc as plsc

DHW = 128
KCONST = (np.pi / 3.5) ** 2
N_ATOMS = 2000
NC, NS, L = 2, 16, 16
NW = NC * NS
NZ = 2
NPASS = DHW // (NW * NZ)
PLANE = DHW * DHW
SLAB = NZ * PLANE
QSTRIDE = 6144
WLCAP = N_ATOMS + L

_offs = sorted(
    [(oy, ox) for oy in range(-2, 3) for ox in range(-2, 3) if oy * oy + ox * ox <= 6],
    key=lambda p: p[0] * p[0] + p[1] * p[1],
)
_oy = np.array([o[0] for o in _offs] + [0] * 11, np.int32)
_ox = np.array([o[1] for o in _offs] + [0] * 11, np.int32)
_r2 = np.array([o[0] ** 2 + o[1] ** 2 for o in _offs] + [999] * 11, np.int64)

ZONES = 2 * NPASS * (N_ATOMS + L)

_ITAB = np.concatenate([
    (_oy[0:16] * DHW + _ox[0:16]).astype(np.int32),
    (_oy[16:32] * DHW + _ox[16:32]).astype(np.int32),
    _oy[0:16], _oy[16:32], _ox[0:16], _ox[16:32],
    np.array([0, ZONES, 2 * ZONES] + [0] * 13, np.int32),
]).astype(np.int32)

_wrow = lambda adz, g: np.where(
    adz * adz + _r2[g * 16:(g + 1) * 16] <= 6,
    np.exp(-KCONST * (adz * adz + _r2[g * 16:(g + 1) * 16].astype(np.float64))),
    0.0).astype(np.float32)
_FTAB = np.concatenate(
    [_wrow(a, g) for a in range(4) for g in range(2)] + [np.zeros(16, np.float32)])


def _sc_body(src_hbm, tgt_hbm, itab_hbm, ftab_hbm, out_hbm,
             sg0, sg1, tg0, tg1, cbuf, wl, itab, ftab, outv):
    cid = lax.axis_index("c")
    sid = lax.axis_index("s")
    wid = sid * NC + cid

    pltpu.sync_copy(itab_hbm, itab)
    pltpu.sync_copy(ftab_hbm, ftab)

    off_v = [itab[pl.ds(0, L)], itab[pl.ds(L, L)]]
    oy_v = [itab[pl.ds(2 * L, L)], itab[pl.ds(3 * L, L)]]
    ox_v = [itab[pl.ds(4 * L, L)], itab[pl.ds(5 * L, L)]]
    wstride_v = itab[pl.ds(6 * L, L)]
    zero_v = ftab[pl.ds(8 * L, L)]

    s0s = [p * (NW * NZ) + wid * NZ for p in range(NPASS)]

    def scan_cloud(hbm, wbase):
        pltpu.sync_copy(hbm, cbuf)

        def sb(i, cnts):
            zv = (cbuf[pl.ds(i * L, L)] * np.float32(DHW)).astype(jnp.int32)
            yv = (cbuf[pl.ds(N_ATOMS + i * L, L)]
                  * np.float32(DHW)).astype(jnp.int32)
            xv = (cbuf[pl.ds(2 * N_ATOMS + i * L, L)]
                  * np.float32(DHW)).astype(jnp.int32)
            new = []
            for p in range(NPASS):
                h = (zv >= s0s[p] - 2) & (zv <= s0s[p] + NZ + 1)
                c = cnts[p]
                at = wbase + p * WLCAP + c
                plsc.store_compressed(wl.at[pl.ds(at, L)], zv, mask=h)
                plsc.store_compressed(wl.at[pl.ds(ZONES + at, L)], yv, mask=h)
                plsc.store_compressed(
                    wl.at[pl.ds(2 * ZONES + at, L)], xv, mask=h)
                new.append(c + plsc.all_reduce_population_count(h)[0])
            return tuple(new)

        return lax.fori_loop(0, N_ATOMS // L, sb, (0,) * NPASS)

    nsrc = scan_cloud(src_hbm, 0)
    ntgt = scan_cloud(tgt_hbm, NPASS * WLCAP)

    def atom_body(g0, g1, w, s0):
        v = plsc.load_gather(wl, [w + wstride_v])
        zc, yc, xc = v[0], v[1], v[2]
        byx = yc * DHW + xc
        myx = []
        for g in range(2):
            y = yc + oy_v[g]
            x = xc + ox_v[g]
            myx.append((y >= 0) & (y < DHW) & (x >= 0) & (x < DHW))
        for lz, grid in ((0, g0), (1, g1)):
            adz = jnp.abs(s0 + lz - zc)

            @pl.when(adz <= 2)
            def _(grid=grid, adz=adz):
                w0 = ftab[pl.ds(adz * 2 * L, L)]
                idx0 = byx + off_v[0]
                cur0 = plsc.load_gather(grid, [idx0], mask=myx[0])
                plsc.store_scatter(
                    grid, [idx0], jnp.maximum(cur0, w0), mask=myx[0])

                @pl.when(adz <= 1)
                def _(grid=grid, adz=adz):
                    w1 = ftab[pl.ds((adz * 2 + 1) * L, L)]
                    idx1 = byx + off_v[1]
                    cur1 = plsc.load_gather(grid, [idx1], mask=myx[1])
                    plsc.store_scatter(
                        grid, [idx1], jnp.maximum(cur1, w1), mask=myx[1])

    def splat_pair(wb_s, ns, wb_t, nt, s0):
        nmin = jnp.minimum(ns, nt)

        def body2(a, _):
            atom_body(sg0, sg1, wb_s + a, s0)
            atom_body(tg0, tg1, wb_t + a, s0)
            return 0

        lax.fori_loop(0, nmin, body2, 0)

        def body_s(a, _):
            atom_body(sg0, sg1, wb_s + a, s0)
            return 0

        lax.fori_loop(nmin, ns, body_s, 0)

        def body_t(a, _):
            atom_body(tg0, tg1, wb_t + a, s0)
            return 0

        lax.fori_loop(nmin, nt, body_t, 0)

    acc_ss = zero_v
    acc_st = zero_v
    for p in range(NPASS):

        def zbody(i, _):
            sg0[pl.ds(i * L, L)] = zero_v
            sg1[pl.ds(i * L, L)] = zero_v
            tg0[pl.ds(i * L, L)] = zero_v
            tg1[pl.ds(i * L, L)] = zero_v
            return 0

        lax.fori_loop(0, PLANE // L, zbody, 0, unroll=8)

        splat_pair(p * WLCAP, nsrc[p],
                   (NPASS + p) * WLCAP, ntgt[p], s0s[p])

        def rbody(i, carry):
            css, cst = carry
            s0v = sg0[pl.ds(i * L, L)]
            s1v = sg1[pl.ds(i * L, L)]
            t0v = tg0[pl.ds(i * L, L)]
            t1v = tg1[pl.ds(i * L, L)]
            return (css + s0v * s0v + s1v * s1v,
                    cst + s0v * t0v + s1v * t1v)

        acc_ss, acc_st = lax.fori_loop(
            0, PLANE // L, rbody, (acc_ss, acc_st), unroll=8)

    for i in range(2 * DHW // L):
        outv[pl.ds(i * L, L)] = zero_v
    outv[pl.ds(0, L)] = acc_ss
    outv[pl.ds(DHW, L)] = acc_st
    pltpu.sync_copy(outv.at[pl.ds(0, DHW)], out_hbm.at[pl.ds(wid * DHW, DHW)])
    pltpu.sync_copy(outv.at[pl.ds(DHW, DHW)],
                    out_hbm.at[pl.ds((NW + wid) * DHW, DHW)])


@jax.jit
def _run(srcc, tgtt):
    mesh = plsc.VectorSubcoreMesh(
        core_axis_name="c", subcore_axis_name="s", num_cores=NC, num_subcores=NS)
    out = pl.kernel(
        _sc_body,
        out_type=jax.ShapeDtypeStruct((2 * NW * DHW,), jnp.float32),
        mesh=mesh,
        compiler_params=pltpu.CompilerParams(needs_layout_passes=False),
        scratch_types=[
            pltpu.VMEM((PLANE,), jnp.float32),
            pltpu.VMEM((PLANE,), jnp.float32),
            pltpu.VMEM((PLANE,), jnp.float32),
            pltpu.VMEM((PLANE,), jnp.float32),
            pltpu.VMEM((3 * N_ATOMS,), jnp.float32),
            pltpu.VMEM((3 * ZONES,), jnp.int32),
            pltpu.VMEM((7 * L,), jnp.int32),
            pltpu.VMEM((9 * L,), jnp.float32),
            pltpu.VMEM((2 * DHW,), jnp.float32),
        ],
    )(srcc, tgtt, jnp.asarray(_ITAB), jnp.asarray(_FTAB))
    halves = out.reshape(2, NW * DHW)
    return jnp.sum(halves[0]) - jnp.sum(halves[1])


def kernel(src, tgt):
    return _run(src.reshape(3 * N_ATOMS), tgt.reshape(3 * N_ATOMS))


# speedup vs baseline: 1.0085x; 1.0085x over previous
import numpy as np
import jax
import jax.numpy as jnp
from jax import lax
from jax.experimental import pallas as pl
from jax.experimental.pallas import tpu as pltpu
from jax.experimental.pallas import tpu_sc as plsc

DHW = 128
KCONST = (np.pi / 3.5) ** 2
N_ATOMS = 2000
NC, NS, L = 2, 16, 16
NW = NC * NS
NZ = 2
NPASS = DHW // (NW * NZ)
PLANE = DHW * DHW
SLAB = NZ * PLANE
QSTRIDE = 6144
WLCAP = N_ATOMS + L

_offs = sorted(
    [(oy, ox) for oy in range(-2, 3) for ox in range(-2, 3) if oy * oy + ox * ox <= 6],
    key=lambda p: p[0] * p[0] + p[1] * p[1],
)
_oy = np.array([o[0] for o in _offs] + [0] * 11, np.int32)
_ox = np.array([o[1] for o in _offs] + [0] * 11, np.int32)
_r2 = np.array([o[0] ** 2 + o[1] ** 2 for o in _offs] + [999] * 11, np.int64)

ZONES = 2 * NPASS * (N_ATOMS + L)

_ITAB = np.concatenate([
    (_oy[0:16] * DHW + _ox[0:16]).astype(np.int32),
    (_oy[16:32] * DHW + _ox[16:32]).astype(np.int32),
    _oy[0:16], _oy[16:32], _ox[0:16], _ox[16:32],
    np.array([0, ZONES, 2 * ZONES] + [0] * 13, np.int32),
]).astype(np.int32)

_wrow = lambda adz, g: np.where(
    adz * adz + _r2[g * 16:(g + 1) * 16] <= 6,
    np.exp(-KCONST * (adz * adz + _r2[g * 16:(g + 1) * 16].astype(np.float64))),
    0.0).astype(np.float32)
_FTAB = np.concatenate(
    [_wrow(a, g) for a in range(4) for g in range(2)] + [np.zeros(16, np.float32)])


def _sc_body(src_hbm, tgt_hbm, itab_hbm, ftab_hbm, out_hbm,
             sg0, sg1, tg0, tg1, cbuf, wl, itab, ftab, outv):
    cid = lax.axis_index("c")
    sid = lax.axis_index("s")
    wid = sid * NC + cid

    pltpu.sync_copy(itab_hbm, itab)
    pltpu.sync_copy(ftab_hbm, ftab)

    off_v = [itab[pl.ds(0, L)], itab[pl.ds(L, L)]]
    oy_v = [itab[pl.ds(2 * L, L)], itab[pl.ds(3 * L, L)]]
    ox_v = [itab[pl.ds(4 * L, L)], itab[pl.ds(5 * L, L)]]
    wstride_v = itab[pl.ds(6 * L, L)]
    zero_v = ftab[pl.ds(8 * L, L)]

    s0s = [p * (NW * NZ) + wid * NZ for p in range(NPASS)]

    def scan_cloud(hbm, wbase):
        pltpu.sync_copy(hbm, cbuf)

        def sb(i, cnts):
            zv = (cbuf[pl.ds(i * L, L)] * np.float32(DHW)).astype(jnp.int32)
            yv = (cbuf[pl.ds(N_ATOMS + i * L, L)]
                  * np.float32(DHW)).astype(jnp.int32)
            xv = (cbuf[pl.ds(2 * N_ATOMS + i * L, L)]
                  * np.float32(DHW)).astype(jnp.int32)
            new = []
            for p in range(NPASS):
                h = (zv >= s0s[p] - 2) & (zv <= s0s[p] + NZ + 1)
                c = cnts[p]
                at = wbase + p * WLCAP + c
                plsc.store_compressed(wl.at[pl.ds(at, L)], zv, mask=h)
                plsc.store_compressed(wl.at[pl.ds(ZONES + at, L)], yv, mask=h)
                plsc.store_compressed(
                    wl.at[pl.ds(2 * ZONES + at, L)], xv, mask=h)
                new.append(c + plsc.all_reduce_population_count(h)[0])
            return tuple(new)

        return lax.fori_loop(0, N_ATOMS // L, sb, (0,) * NPASS)

    nsrc = scan_cloud(src_hbm, 0)
    ntgt = scan_cloud(tgt_hbm, NPASS * WLCAP)

    def atom_body(g0, g1, w, s0):
        v = plsc.load_gather(wl, [w + wstride_v])
        zc, yc, xc = v[0], v[1], v[2]
        byx = yc * DHW + xc
        myx = []
        for g in range(2):
            y = yc + oy_v[g]
            x = xc + ox_v[g]
            myx.append((y >= 0) & (y < DHW) & (x >= 0) & (x < DHW))
        for lz, grid in ((0, g0), (1, g1)):
            adz = jnp.abs(s0 + lz - zc)

            @pl.when(adz <= 2)
            def _(grid=grid, adz=adz):
                w0 = ftab[pl.ds(adz * 2 * L, L)]
                idx0 = byx + off_v[0]
                cur0 = plsc.load_gather(grid, [idx0], mask=myx[0])
                plsc.store_scatter(
                    grid, [idx0], jnp.maximum(cur0, w0), mask=myx[0])

                @pl.when(adz <= 1)
                def _(grid=grid, adz=adz):
                    w1 = ftab[pl.ds((adz * 2 + 1) * L, L)]
                    idx1 = byx + off_v[1]
                    cur1 = plsc.load_gather(grid, [idx1], mask=myx[1])
                    plsc.store_scatter(
                        grid, [idx1], jnp.maximum(cur1, w1), mask=myx[1])

    def splat_pair(wb_s, ns, wb_t, nt, s0):
        def body_s(a, _):
            atom_body(sg0, sg1, wb_s + a, s0)
            return 0

        lax.fori_loop(0, ns, body_s, 0)

        def body_t(a, _):
            atom_body(tg0, tg1, wb_t + a, s0)
            return 0

        lax.fori_loop(0, nt, body_t, 0)

    acc_ss = zero_v
    acc_st = zero_v
    for p in range(NPASS):

        def zbody(i, _):
            sg0[pl.ds(i * L, L)] = zero_v
            sg1[pl.ds(i * L, L)] = zero_v
            tg0[pl.ds(i * L, L)] = zero_v
            tg1[pl.ds(i * L, L)] = zero_v
            return 0

        lax.fori_loop(0, PLANE // L, zbody, 0, unroll=8)

        splat_pair(p * WLCAP, nsrc[p],
                   (NPASS + p) * WLCAP, ntgt[p], s0s[p])

        def rbody(i, carry):
            css, cst = carry
            s0v = sg0[pl.ds(i * L, L)]
            s1v = sg1[pl.ds(i * L, L)]
            t0v = tg0[pl.ds(i * L, L)]
            t1v = tg1[pl.ds(i * L, L)]
            return (css + s0v * s0v + s1v * s1v,
                    cst + s0v * t0v + s1v * t1v)

        acc_ss, acc_st = lax.fori_loop(
            0, PLANE // L, rbody, (acc_ss, acc_st), unroll=8)

    for i in range(2 * DHW // L):
        outv[pl.ds(i * L, L)] = zero_v
    outv[pl.ds(0, L)] = acc_ss
    outv[pl.ds(DHW, L)] = acc_st
    pltpu.sync_copy(outv.at[pl.ds(0, DHW)], out_hbm.at[pl.ds(wid * DHW, DHW)])
    pltpu.sync_copy(outv.at[pl.ds(DHW, DHW)],
                    out_hbm.at[pl.ds((NW + wid) * DHW, DHW)])


@jax.jit
def _run(srcc, tgtt):
    mesh = plsc.VectorSubcoreMesh(
        core_axis_name="c", subcore_axis_name="s", num_cores=NC, num_subcores=NS)
    out = pl.kernel(
        _sc_body,
        out_type=jax.ShapeDtypeStruct((2 * NW * DHW,), jnp.float32),
        mesh=mesh,
        compiler_params=pltpu.CompilerParams(needs_layout_passes=False),
        scratch_types=[
            pltpu.VMEM((PLANE,), jnp.float32),
            pltpu.VMEM((PLANE,), jnp.float32),
            pltpu.VMEM((PLANE,), jnp.float32),
            pltpu.VMEM((PLANE,), jnp.float32),
            pltpu.VMEM((3 * N_ATOMS,), jnp.float32),
            pltpu.VMEM((3 * ZONES,), jnp.int32),
            pltpu.VMEM((7 * L,), jnp.int32),
            pltpu.VMEM((9 * L,), jnp.float32),
            pltpu.VMEM((2 * DHW,), jnp.float32),
        ],
    )(srcc, tgtt, jnp.asarray(_ITAB), jnp.asarray(_FTAB))
    halves = out.reshape(2, NW * DHW)
    return jnp.sum(halves[0]) - jnp.sum(halves[1])


def kernel(src, tgt):
    return _run(src.reshape(3 * N_ATOMS), tgt.reshape(3 * N_ATOMS))


# speedup vs baseline: 1.0680x; 1.0591x over previous
import numpy as np
import jax
import jax.numpy as jnp
from jax import lax
from jax.experimental import pallas as pl
from jax.experimental.pallas import tpu as pltpu
from jax.experimental.pallas import tpu_sc as plsc

DHW = 128
KCONST = (np.pi / 3.5) ** 2
N_ATOMS = 2000
NC, NS, L = 2, 16, 16
NW = NC * NS
NZ = 2
NPASS = DHW // (NW * NZ)
PLANE = DHW * DHW
SLAB = NZ * PLANE
QSTRIDE = 6144
WLCAP = N_ATOMS + L

_offs = sorted(
    [(oy, ox) for oy in range(-2, 3) for ox in range(-2, 3) if oy * oy + ox * ox <= 6],
    key=lambda p: p[0] * p[0] + p[1] * p[1],
)
_oy = np.array([o[0] for o in _offs] + [0] * 11, np.int32)
_ox = np.array([o[1] for o in _offs] + [0] * 11, np.int32)
_r2 = np.array([o[0] ** 2 + o[1] ** 2 for o in _offs] + [999] * 11, np.int64)

ZONES = 2 * NPASS * (N_ATOMS + L)

_ITAB = np.concatenate([
    (_oy[0:16] * DHW + _ox[0:16]).astype(np.int32),
    (_oy[16:32] * DHW + _ox[16:32]).astype(np.int32),
    _oy[0:16], _oy[16:32], _ox[0:16], _ox[16:32],
    np.array([0, ZONES, 2 * ZONES] + [0] * 13, np.int32),
]).astype(np.int32)

_wrow = lambda adz, g: np.where(
    adz * adz + _r2[g * 16:(g + 1) * 16] <= 6,
    np.exp(-KCONST * (adz * adz + _r2[g * 16:(g + 1) * 16].astype(np.float64))),
    0.0).astype(np.float32)
_FTAB = np.concatenate(
    [_wrow(a, g) for a in range(4) for g in range(2)] + [np.zeros(16, np.float32)])


def _sc_body(src_hbm, tgt_hbm, itab_hbm, ftab_hbm, out_hbm,
             sg0, sg1, tg0, tg1, cbuf, wl, itab, ftab, outv):
    cid = lax.axis_index("c")
    sid = lax.axis_index("s")
    wid = sid * NC + cid

    pltpu.sync_copy(itab_hbm, itab)
    pltpu.sync_copy(ftab_hbm, ftab)

    off_v = [itab[pl.ds(0, L)], itab[pl.ds(L, L)]]
    oy_v = [itab[pl.ds(2 * L, L)], itab[pl.ds(3 * L, L)]]
    ox_v = [itab[pl.ds(4 * L, L)], itab[pl.ds(5 * L, L)]]
    wstride_v = itab[pl.ds(6 * L, L)]
    zero_v = ftab[pl.ds(8 * L, L)]

    s0s = [p * (NW * NZ) + wid * NZ for p in range(NPASS)]

    def scan_cloud(hbm, wbase):
        pltpu.sync_copy(hbm, cbuf)

        def sb(i, cnts):
            zv = (cbuf[pl.ds(i * L, L)] * np.float32(DHW)).astype(jnp.int32)
            yv = (cbuf[pl.ds(N_ATOMS + i * L, L)]
                  * np.float32(DHW)).astype(jnp.int32)
            xv = (cbuf[pl.ds(2 * N_ATOMS + i * L, L)]
                  * np.float32(DHW)).astype(jnp.int32)
            new = []
            for p in range(NPASS):
                h = (zv >= s0s[p] - 2) & (zv <= s0s[p] + NZ + 1)
                c = cnts[p]
                at = wbase + p * WLCAP + c
                plsc.store_compressed(wl.at[pl.ds(at, L)], zv, mask=h)
                plsc.store_compressed(wl.at[pl.ds(ZONES + at, L)], yv, mask=h)
                plsc.store_compressed(
                    wl.at[pl.ds(2 * ZONES + at, L)], xv, mask=h)
                new.append(c + plsc.all_reduce_population_count(h)[0])
            return tuple(new)

        return lax.fori_loop(0, N_ATOMS // L, sb, (0,) * NPASS)

    nsrc = scan_cloud(src_hbm, 0)
    ntgt = scan_cloud(tgt_hbm, NPASS * WLCAP)

    def atom_body(g0, g1, w, s0):
        zc = wl[pl.ds(w, L)][0]
        yc = wl[pl.ds(ZONES + w, L)][0]
        xc = wl[pl.ds(2 * ZONES + w, L)][0]
        byx = yc * DHW + xc
        myx = []
        for g in range(2):
            y = yc + oy_v[g]
            x = xc + ox_v[g]
            myx.append((y >= 0) & (y < DHW) & (x >= 0) & (x < DHW))
        for lz, grid in ((0, g0), (1, g1)):
            adz = jnp.abs(s0 + lz - zc)

            @pl.when(adz <= 2)
            def _(grid=grid, adz=adz):
                w0 = ftab[pl.ds(adz * 2 * L, L)]
                idx0 = byx + off_v[0]
                cur0 = plsc.load_gather(grid, [idx0], mask=myx[0])
                plsc.store_scatter(
                    grid, [idx0], jnp.maximum(cur0, w0), mask=myx[0])

                @pl.when(adz <= 1)
                def _(grid=grid, adz=adz):
                    w1 = ftab[pl.ds((adz * 2 + 1) * L, L)]
                    idx1 = byx + off_v[1]
                    cur1 = plsc.load_gather(grid, [idx1], mask=myx[1])
                    plsc.store_scatter(
                        grid, [idx1], jnp.maximum(cur1, w1), mask=myx[1])

    def splat_pair(wb_s, ns, wb_t, nt, s0):
        def body_s(a, _):
            atom_body(sg0, sg1, wb_s + a, s0)
            return 0

        lax.fori_loop(0, ns, body_s, 0)

        def body_t(a, _):
            atom_body(tg0, tg1, wb_t + a, s0)
            return 0

        lax.fori_loop(0, nt, body_t, 0)

    acc_ss = zero_v
    acc_st = zero_v
    for p in range(NPASS):

        def zbody(i, _):
            sg0[pl.ds(i * L, L)] = zero_v
            sg1[pl.ds(i * L, L)] = zero_v
            tg0[pl.ds(i * L, L)] = zero_v
            tg1[pl.ds(i * L, L)] = zero_v
            return 0

        lax.fori_loop(0, PLANE // L, zbody, 0, unroll=8)

        splat_pair(p * WLCAP, nsrc[p],
                   (NPASS + p) * WLCAP, ntgt[p], s0s[p])

        def rbody(i, carry):
            css, cst = carry
            s0v = sg0[pl.ds(i * L, L)]
            s1v = sg1[pl.ds(i * L, L)]
            t0v = tg0[pl.ds(i * L, L)]
            t1v = tg1[pl.ds(i * L, L)]
            return (css + s0v * s0v + s1v * s1v,
                    cst + s0v * t0v + s1v * t1v)

        acc_ss, acc_st = lax.fori_loop(
            0, PLANE // L, rbody, (acc_ss, acc_st), unroll=8)

    for i in range(2 * DHW // L):
        outv[pl.ds(i * L, L)] = zero_v
    outv[pl.ds(0, L)] = acc_ss
    outv[pl.ds(DHW, L)] = acc_st
    pltpu.sync_copy(outv.at[pl.ds(0, DHW)], out_hbm.at[pl.ds(wid * DHW, DHW)])
    pltpu.sync_copy(outv.at[pl.ds(DHW, DHW)],
                    out_hbm.at[pl.ds((NW + wid) * DHW, DHW)])


@jax.jit
def _run(srcc, tgtt):
    mesh = plsc.VectorSubcoreMesh(
        core_axis_name="c", subcore_axis_name="s", num_cores=NC, num_subcores=NS)
    out = pl.kernel(
        _sc_body,
        out_type=jax.ShapeDtypeStruct((2 * NW * DHW,), jnp.float32),
        mesh=mesh,
        compiler_params=pltpu.CompilerParams(needs_layout_passes=False),
        scratch_types=[
            pltpu.VMEM((PLANE,), jnp.float32),
            pltpu.VMEM((PLANE,), jnp.float32),
            pltpu.VMEM((PLANE,), jnp.float32),
            pltpu.VMEM((PLANE,), jnp.float32),
            pltpu.VMEM((3 * N_ATOMS,), jnp.float32),
            pltpu.VMEM((3 * ZONES,), jnp.int32),
            pltpu.VMEM((7 * L,), jnp.int32),
            pltpu.VMEM((9 * L,), jnp.float32),
            pltpu.VMEM((2 * DHW,), jnp.float32),
        ],
    )(srcc, tgtt, jnp.asarray(_ITAB), jnp.asarray(_FTAB))
    halves = out.reshape(2, NW * DHW)
    return jnp.sum(halves[0]) - jnp.sum(halves[1])


def kernel(src, tgt):
    return _run(src.reshape(3 * N_ATOMS), tgt.reshape(3 * N_ATOMS))
